# baseline (speedup 1.0000x reference)
import jax
import jax.numpy as jnp
from jax import lax
from jax.experimental import pallas as pl
from jax.experimental.pallas import tpu as pltpu, tpu_sc as plsc

H = 384
W = 384
N = 64
KSZ = 11
RAD = 5
NC = 2
NS = 16
NW = NC * NS
SPW = N // NW
ZROWS = 32
NZ = H // ZROWS
SROWS = 24
STRIPROW = 32
WRAPROW = STRIPROW + SPW * SROWS
ZBROWS = WRAPROW + SROWS
NVREG = 8


def _body(combo_hbm, out_hbm, pts_v, kern_v, zbuf, psem, ksem, zsem, ssem):
    c = lax.axis_index("c")
    s = lax.axis_index("s")
    w = s * NC + c

    cp_p = pltpu.async_copy(combo_hbm.at[w], pts_v, psem)
    cp_k = pltpu.async_copy(combo_hbm.at[pl.ds(NW, NVREG)], kern_v, ksem)

    zero16 = jnp.zeros((16,), jnp.float32)

    def _zero_flat(i, carry):
        r = lax.div(i, W // 16)
        col = (i - r * (W // 16)) * 16
        zbuf[r, pl.ds(col, 16)] = zero16
        return carry

    lax.fori_loop(0, ZROWS * (W // 16), _zero_flat, 0, unroll=8)

    zcopies = [
        pltpu.async_copy(
            zbuf.at[pl.ds(0, ZROWS)],
            out_hbm.at[w * SPW + si, pl.ds(i * ZROWS, ZROWS)],
            zsem,
        )
        for si in range(SPW)
        for i in range(NZ)
    ]

    lax.fori_loop(
        ZROWS * (W // 16), ZBROWS * (W // 16), _zero_flat, 0, unroll=8
    )

    cp_p.wait()
    cp_k.wait()

    lane = lax.broadcasted_iota(jnp.int32, (16,), 0)
    pv = pts_v[...]

    def _scalar_at(i):
        return jnp.clip(pv[i].astype(jnp.int32), 0, W - 1)

    cxs = [_scalar_at(2 * si) for si in range(SPW)]
    cys = [_scalar_at(2 * si + 1) for si in range(SPW)]
    yws = [
        pl.multiple_of(jnp.clip(lax.div(cy - RAD, 8) * 8, 0, H - SROWS), 8)
        for cy in cys
    ]

    kys, kxs, kms, kvs = [], [], [], []
    for j in range(NVREG):
        t = lane + j * 16
        ky = lax.div(t, KSZ)
        kys.append(ky)
        kxs.append(t - ky * KSZ)
        kms.append(t < KSZ * KSZ)
        kvs.append(kern_v[j, :])

    for cp in zcopies:
        cp.wait()

    wraps = []
    for si in range(SPW):
        cx, cy, yw = cxs[si], cys[si], yws[si]
        rows, cols, wmasks = [], [], []
        for j in range(NVREG):
            yy = kys[j] + (cy - RAD)
            xx = kxs[j] + (cx - RAD)
            xxw = jnp.where(xx < 0, xx + W, xx)
            m = kms[j] & (yy >= 0) & (yy < H) & (xx < W)
            m2 = kms[j] & (yy < 0) & (xx < W)
            rows.append(
                jnp.where(
                    m,
                    yy - yw + STRIPROW + si * SROWS,
                    STRIPROW + si * SROWS,
                )
            )
            cols.append(jnp.where(m | m2, xxw, 0))
            wmasks.append(m2)
            plsc.store_scatter(zbuf, [rows[j], cols[j]], kvs[j], mask=m)
        wraps.append((rows, cols, wmasks))

    scopies = [
        pltpu.async_copy(
            zbuf.at[pl.ds(STRIPROW + si * SROWS, SROWS)],
            out_hbm.at[w * SPW + si, pl.ds(yws[si], SROWS)],
            ssem,
        )
        for si in range(SPW)
    ]
    for cp in scopies:
        cp.wait()

    for si in range(SPW):
        cy = cys[si]

        @pl.when(cy < RAD)
        def _wrap_rows(si=si, cy=cy):
            _, cols, wmasks = wraps[si]
            wrows = []
            for j in range(NVREG):
                yy = kys[j] + (cy - RAD)
                wrows.append(
                    jnp.where(wmasks[j], yy + SROWS + WRAPROW, WRAPROW)
                )
                plsc.store_scatter(
                    zbuf, [wrows[j], cols[j]], kvs[j], mask=wmasks[j]
                )
            pltpu.sync_copy(
                zbuf.at[pl.ds(WRAPROW, SROWS)],
                out_hbm.at[w * SPW + si, pl.ds(H - SROWS, SROWS)],
            )
            for j in range(NVREG):
                plsc.store_scatter(
                    zbuf, [wrows[j], cols[j]], zero16, mask=wmasks[j]
                )


@jax.jit
def _heatmap_sc(combo):
    mesh = plsc.VectorSubcoreMesh(
        core_axis_name="c", subcore_axis_name="s", num_cores=NC, num_subcores=NS
    )
    run = pl.kernel(
        _body,
        out_type=jax.ShapeDtypeStruct((N, H, W), jnp.float32),
        mesh=mesh,
        scratch_types=[
            pltpu.VMEM((16,), jnp.float32),
            pltpu.VMEM((NVREG, 16), jnp.float32),
            pltpu.VMEM((ZBROWS, W), jnp.float32),
            pltpu.SemaphoreType.DMA,
            pltpu.SemaphoreType.DMA,
            pltpu.SemaphoreType.DMA,
            pltpu.SemaphoreType.DMA,
        ],
        compiler_params=pltpu.CompilerParams(needs_layout_passes=False),
    )
    return run(combo)


def kernel(points, img, kernel):
    pts_part = jnp.pad(points.reshape(NW, 2 * SPW), ((0, 0), (0, 16 - 2 * SPW)))
    kern_part = jnp.pad(kernel.reshape(-1), (0, NVREG * 16 - KSZ * KSZ))
    combo = jnp.concatenate(
        [pts_part, kern_part.reshape(NVREG, 16).astype(jnp.float32)], axis=0
    )
    return _heatmap_sc(combo)


# speedup vs baseline: 1.0277x; 1.0277x over previous
import jax
import jax.numpy as jnp
from jax import lax
from jax.experimental import pallas as pl
from jax.experimental.pallas import tpu as pltpu, tpu_sc as plsc

H = 384
W = 384
N = 64
KSZ = 11
RAD = 5
NC = 2
NS = 16
NW = NC * NS
SPW = N // NW
ZROWS = 64
NZ = H // ZROWS
SROWS = 24
STRIPROW = 0
WRAPROW = 48
ZBROWS = WRAPROW + SROWS
NVREG = 8


def _body(combo_hbm, out_hbm, pts_v, kern_v, zbuf, psem, ksem, zsem, ssem):
    c = lax.axis_index("c")
    s = lax.axis_index("s")
    w = s * NC + c

    cp_p = pltpu.async_copy(combo_hbm.at[w], pts_v, psem)
    cp_k = pltpu.async_copy(combo_hbm.at[pl.ds(NW, NVREG)], kern_v, ksem)

    zero16 = jnp.zeros((16,), jnp.float32)

    def _zero_flat(i, carry):
        r = lax.div(i, W // 16)
        col = (i - r * (W // 16)) * 16
        zbuf[r, pl.ds(col, 16)] = zero16
        return carry

    lax.fori_loop(0, ZBROWS * (W // 16), _zero_flat, 0, unroll=8)

    zcopies = [
        pltpu.async_copy(
            zbuf.at[pl.ds(0, ZROWS)],
            out_hbm.at[w * SPW + si, pl.ds(i * ZROWS, ZROWS)],
            zsem,
        )
        for si in range(SPW)
        for i in range(NZ)
    ]

    cp_p.wait()
    cp_k.wait()

    lane = lax.broadcasted_iota(jnp.int32, (16,), 0)
    pv = pts_v[...]

    def _scalar_at(i):
        return jnp.clip(pv[i].astype(jnp.int32), 0, W - 1)

    cxs = [_scalar_at(2 * si) for si in range(SPW)]
    cys = [_scalar_at(2 * si + 1) for si in range(SPW)]
    yws = [
        pl.multiple_of(jnp.clip(lax.div(cy - RAD, 8) * 8, 0, H - SROWS), 8)
        for cy in cys
    ]

    kys, kxs, kms, kvs = [], [], [], []
    for j in range(NVREG):
        t = lane + j * 16
        ky = lax.div(t, KSZ)
        kys.append(ky)
        kxs.append(t - ky * KSZ)
        kms.append(t < KSZ * KSZ)
        kvs.append(kern_v[j, :])

    for cp in zcopies:
        cp.wait()

    wraps = []
    for si in range(SPW):
        cx, cy, yw = cxs[si], cys[si], yws[si]
        rows, cols, wmasks = [], [], []
        for j in range(NVREG):
            yy = kys[j] + (cy - RAD)
            xx = kxs[j] + (cx - RAD)
            xxw = jnp.where(xx < 0, xx + W, xx)
            m = kms[j] & (yy >= 0) & (yy < H) & (xx < W)
            m2 = kms[j] & (yy < 0) & (xx < W)
            rows.append(
                jnp.where(
                    m,
                    yy - yw + STRIPROW + si * SROWS,
                    STRIPROW + si * SROWS,
                )
            )
            cols.append(jnp.where(m | m2, xxw, 0))
            wmasks.append(m2)
            plsc.store_scatter(zbuf, [rows[j], cols[j]], kvs[j], mask=m)
        wraps.append((rows, cols, wmasks))

    scopies = [
        pltpu.async_copy(
            zbuf.at[pl.ds(STRIPROW + si * SROWS, SROWS)],
            out_hbm.at[w * SPW + si, pl.ds(yws[si], SROWS)],
            ssem,
        )
        for si in range(SPW)
    ]
    for cp in scopies:
        cp.wait()

    for si in range(SPW):
        cy = cys[si]

        @pl.when(cy < RAD)
        def _wrap_rows(si=si, cy=cy):
            _, cols, wmasks = wraps[si]
            wrows = []
            for j in range(NVREG):
                yy = kys[j] + (cy - RAD)
                wrows.append(
                    jnp.where(wmasks[j], yy + SROWS + WRAPROW, WRAPROW)
                )
                plsc.store_scatter(
                    zbuf, [wrows[j], cols[j]], kvs[j], mask=wmasks[j]
                )
            pltpu.sync_copy(
                zbuf.at[pl.ds(WRAPROW, SROWS)],
                out_hbm.at[w * SPW + si, pl.ds(H - SROWS, SROWS)],
            )
            for j in range(NVREG):
                plsc.store_scatter(
                    zbuf, [wrows[j], cols[j]], zero16, mask=wmasks[j]
                )


@jax.jit
def _heatmap_sc(combo):
    mesh = plsc.VectorSubcoreMesh(
        core_axis_name="c", subcore_axis_name="s", num_cores=NC, num_subcores=NS
    )
    run = pl.kernel(
        _body,
        out_type=jax.ShapeDtypeStruct((N, H, W), jnp.float32),
        mesh=mesh,
        scratch_types=[
            pltpu.VMEM((16,), jnp.float32),
            pltpu.VMEM((NVREG, 16), jnp.float32),
            pltpu.VMEM((ZBROWS, W), jnp.float32),
            pltpu.SemaphoreType.DMA,
            pltpu.SemaphoreType.DMA,
            pltpu.SemaphoreType.DMA,
            pltpu.SemaphoreType.DMA,
        ],
        compiler_params=pltpu.CompilerParams(needs_layout_passes=False),
    )
    return run(combo)


def kernel(points, img, kernel):
    pts_part = jnp.pad(points.reshape(NW, 2 * SPW), ((0, 0), (0, 16 - 2 * SPW)))
    kern_part = jnp.pad(kernel.reshape(-1), (0, NVREG * 16 - KSZ * KSZ))
    combo = jnp.concatenate(
        [pts_part, kern_part.reshape(NVREG, 16).astype(jnp.float32)], axis=0
    )
    return _heatmap_sc(combo)
